# Initial kernel scaffold; baseline (speedup 1.0000x reference)
import jax
import jax.numpy as jnp
from jax.experimental import pallas as pl


def kernel(x, codebook):
    raise NotImplementedError("write your pallas kernel here")


# speedup vs baseline: 1.4062x; 1.4062x over previous
import functools

import jax
import jax.numpy as jnp
from jax import lax
from jax.experimental import pallas as pl
from jax.experimental.pallas import tpu as pltpu
from jax.experimental.pallas import tpu_sc as plsc

_NB_CODE = 8192
_CODE_DIM = 32
_TOK_BLK = 256


def _dist_argmin_body(x_ref, kw_ref, idx_ref, loss_ref, *, inv_count):
    xb = x_ref[...]
    kw = kw_ref[...]
    mm = lax.dot_general(xb, kw, (((1,), (0,)), ((), ())),
                         preferred_element_type=jnp.float32)
    x2 = jnp.sum(xb * xb, axis=1, keepdims=True)
    c2 = jnp.sum(kw * kw, axis=0, keepdims=True)
    dmat = x2 - 2.0 * mm + c2
    m = jnp.min(dmat, axis=1, keepdims=True)
    kiota = lax.broadcasted_iota(jnp.int32, dmat.shape, 1)
    idx = jnp.min(jnp.where(dmat == m, kiota, _NB_CODE),
                  axis=1, keepdims=True)
    idx_ref[...] = idx

    prev = jnp.where(pl.program_id(0) == 0, 0.0, loss_ref[...])
    loss_ref[...] = prev + jnp.sum(m)

    @pl.when(pl.program_id(0) == pl.num_programs(0) - 1)
    def _scale():
        loss_ref[...] = loss_ref[...] * inv_count


def _dist_argmin(xf, k_w):
    n_tok = xf.shape[0]
    n_blk = n_tok // _TOK_BLK
    inv_count = 1.0 / (n_tok * _CODE_DIM)
    return pl.pallas_call(
        functools.partial(_dist_argmin_body, inv_count=inv_count),
        grid=(n_blk,),
        in_specs=[
            pl.BlockSpec((_TOK_BLK, _CODE_DIM), lambda i: (i, 0)),
            pl.BlockSpec((_CODE_DIM, _NB_CODE), lambda i: (0, 0)),
        ],
        out_specs=[
            pl.BlockSpec((_TOK_BLK, 1), lambda i: (i, 0)),
            pl.BlockSpec((1, 1), lambda i: (0, 0)),
        ],
        out_shape=[
            jax.ShapeDtypeStruct((n_tok, 1), jnp.int32),
            jax.ShapeDtypeStruct((1, 1), jnp.float32),
        ],
    )(xf, k_w)


def _sc_gather(codebook, idx):
    info = plsc.get_sparse_core_info()
    nc, ns = info.num_cores, info.num_subcores
    nw = nc * ns
    b = idx.shape[0]
    b_per_w = b // nw
    chunk = 128
    n_chunks = b_per_w // chunk
    idx3 = idx.reshape(nw, n_chunks, chunk)
    mesh = plsc.VectorSubcoreMesh(core_axis_name="c", subcore_axis_name="s")

    @functools.partial(
        pl.kernel,
        mesh=mesh,
        compiler_params=pltpu.CompilerParams(use_tc_tiling_on_sc=False),
        out_type=jax.ShapeDtypeStruct((b, _CODE_DIM), jnp.float32),
        scratch_types=[
            pltpu.VMEM((n_chunks, chunk), jnp.int32),
            pltpu.VMEM((b_per_w, _CODE_DIM), jnp.float32),
            pltpu.SemaphoreType.DMA,
        ],
    )
    def gk(table_hbm, idx_hbm, out_hbm, idx_v, rows_v, sem):
        wid = lax.axis_index("s") * nc + lax.axis_index("c")
        base = wid * b_per_w
        pltpu.sync_copy(idx_hbm.at[wid], idx_v)
        copies = []
        for j in range(n_chunks):
            copies.append(pltpu.async_copy(
                table_hbm.at[idx_v.at[j]],
                rows_v.at[pl.ds(j * chunk, chunk)],
                sem))
        for c in copies:
            c.wait()
        pltpu.sync_copy(rows_v, out_hbm.at[pl.ds(base, b_per_w)])

    return gk(codebook, idx3)


def kernel(x, codebook):
    n, t, c = x.shape
    xf = x.reshape(n * t, c)
    idx2, loss = _dist_argmin(xf, codebook.T)
    idx = idx2.reshape(n * t)
    x_d = _sc_gather(codebook, idx)
    return x_d.reshape(n, t, c), loss[0, 0]
